# baseline (speedup 1.0000x reference)
import functools

import jax
import jax.numpy as jnp
from jax import lax
from jax.experimental import pallas as pl
from jax.experimental.pallas import tpu as pltpu
from jax.experimental.pallas import tpu_sc as plsc

N = 10000
D = 128
E = 320000

NC = 2
NS = 16
NW = NC * NS

C = 64
NB = 4
NCH = 160
EP = NCH * C
EPAD = EP * NW

NP = 10240
ZR = NP // NS

_mesh = plsc.VectorSubcoreMesh(core_axis_name="c", subcore_axis_name="s")
_sc_params = pltpu.CompilerParams(use_tc_tiling_on_sc=False)


def _fill(ref, nrows, ncols, value):
    v = jnp.full((16,), value, jnp.float32)

    def body(i, carry):
        for c in range(ncols // 16):
            ref[i, pl.ds(c * 16, 16)] = v
        return carry

    lax.fori_loop(0, nrows, body, 0)


@functools.partial(
    pl.kernel,
    out_type=jax.ShapeDtypeStruct((NC, NP, 16), jnp.float32),
    mesh=_mesh,
    scratch_types=[
        pltpu.VMEM((NCH, C), jnp.int32),
        pltpu.VMEM((C, 16), jnp.float32),
        pltpu.VMEM_SHARED((NP, 16), jnp.float32),
    ],
    compiler_params=_sc_params,
)
def _sc_deg(dst_hbm, out_hbm, didx, ones_v, acc):
    cid = lax.axis_index("c")
    sid = lax.axis_index("s")
    wid = cid * NS + sid

    _fill(ones_v, C, 16, 0.0)
    zbase = sid * ZR
    for k in range(ZR // C):
        pltpu.sync_copy(ones_v, acc.at[pl.ds(zbase + k * C, C)])
    _fill(ones_v, C, 16, 1.0)
    pltpu.sync_copy(dst_hbm.at[pl.ds(wid * NCH, NCH)], didx)
    plsc.subcore_barrier()

    def chunk(i, carry):
        pltpu.sync_copy(ones_v, acc.at[didx.at[i]], add=True)
        return carry

    lax.fori_loop(0, NCH, chunk, 0)
    plsc.subcore_barrier()

    pltpu.sync_copy(acc.at[pl.ds(zbase, ZR)],
                    out_hbm.at[cid, pl.ds(zbase, ZR)])


@functools.partial(
    pl.kernel,
    out_type=jax.ShapeDtypeStruct((NC, NP, D), jnp.float32),
    mesh=_mesh,
    scratch_types=[
        pltpu.VMEM((EP,), jnp.int32),
        [pltpu.VMEM((1, C), jnp.int32) for _ in range(NB)],
        [pltpu.VMEM((C, D), jnp.float32) for _ in range(NB)],
        pltpu.VMEM_SHARED((NP, D), jnp.float32),
        [pltpu.SemaphoreType.DMA for _ in range(NB)],
        [pltpu.SemaphoreType.DMA for _ in range(NB)],
    ],
    compiler_params=_sc_params,
)
def _sc_agg(g_hbm, src_hbm, dst_hbm, out_hbm, sidx, didxs, rows,
            acc, sems, dsems):
    cid = lax.axis_index("c")
    sid = lax.axis_index("s")
    wid = cid * NS + sid

    _fill(rows[0], C, D, 0.0)
    zbase = sid * ZR
    for k in range(ZR // C):
        pltpu.sync_copy(rows[0], acc.at[pl.ds(zbase + k * C, C)])
    pltpu.sync_copy(src_hbm.at[pl.ds(wid * EP, EP)], sidx)
    plsc.subcore_barrier()

    def gather(c, q):
        pltpu.async_copy(g_hbm.at[sidx.at[pl.ds(c * C, C)]], rows[q], sems[q])

    def gather_wait(c, q):
        pltpu.make_async_copy(g_hbm.at[sidx.at[pl.ds(c * C, C)]], rows[q],
                              sems[q]).wait()

    def didx_load(c, q):
        pltpu.async_copy(dst_hbm.at[c + wid * NCH], didxs[q].at[0], dsems[q])

    def didx_wait(c, q):
        pltpu.make_async_copy(dst_hbm.at[c + wid * NCH], didxs[q].at[0],
                              dsems[q]).wait()

    for q in range(NB):
        didx_load(q, q)
        gather(q, q)

    def rounds(p, carry):
        c0 = NB * p
        for q in range(NB):
            c = c0 + q
            gather_wait(c, q)
            didx_wait(c, q)
            pltpu.sync_copy(rows[q], acc.at[didxs[q].at[0]], add=True)
            nxt = jnp.minimum(c + NB, NCH - 1)
            didx_load(nxt, q)
            gather(nxt, q)
        return carry

    lax.fori_loop(0, NCH // NB, rounds, 0)
    for q in range(NB):
        gather_wait(NCH - 1, q)
        didx_wait(NCH - 1, q)
    plsc.subcore_barrier()

    pltpu.sync_copy(acc.at[pl.ds(zbase, ZR)],
                    out_hbm.at[cid, pl.ds(zbase, ZR)])


_RB = 1000


def _dis_of(dref):
    deg = dref[0, :, 0:1] + dref[1, :, 0:1] + 1.0
    return lax.rsqrt(deg)


def _tc1_body(x_ref, w_ref, d_ref, o_ref):
    dis = _dis_of(d_ref)
    h = jnp.dot(x_ref[...], w_ref[...], preferred_element_type=jnp.float32)
    o_ref[...] = h * dis


def _tc2_body(a_ref, g_ref, d_ref, w_ref, b_ref, o_ref):
    dis = _dis_of(d_ref)
    s = a_ref[0] + a_ref[1] + g_ref[...]
    r = jnp.maximum(s * dis + b_ref[...], 0.0)
    o_ref[...] = jnp.dot(r, w_ref[...], preferred_element_type=jnp.float32) * dis


def _tc3_body(a_ref, g_ref, d_ref, b_ref, o_ref):
    dis = _dis_of(d_ref)
    s = a_ref[0] + a_ref[1] + g_ref[...]
    o_ref[...] = s * dis + b_ref[...]


def _row_spec(i):
    return (i, 0)


_spec_rows = pl.BlockSpec((_RB, D), _row_spec)
_spec_acc = pl.BlockSpec((NC, _RB, D), lambda i: (0, i, 0))
_spec_deg = pl.BlockSpec((NC, _RB, 16), lambda i: (0, i, 0))
_spec_w = pl.BlockSpec((D, D), lambda i: (0, 0))
_spec_b = pl.BlockSpec((1, D), lambda i: (0, 0))

_GRID = (N // _RB,)
_out_rows = jax.ShapeDtypeStruct((N, D), jnp.float32)

_tc1 = pl.pallas_call(
    _tc1_body, grid=_GRID,
    in_specs=[_spec_rows, _spec_w, _spec_deg],
    out_specs=_spec_rows, out_shape=_out_rows)

_tc2 = pl.pallas_call(
    _tc2_body, grid=_GRID,
    in_specs=[_spec_acc, _spec_rows, _spec_deg, _spec_w, _spec_b],
    out_specs=_spec_rows, out_shape=_out_rows)

_tc3 = pl.pallas_call(
    _tc3_body, grid=_GRID,
    in_specs=[_spec_acc, _spec_rows, _spec_deg, _spec_b],
    out_specs=_spec_rows, out_shape=_out_rows)


def kernel(x, edge_index, W1, b1, W2, b2):
    ei = edge_index.astype(jnp.int32)
    pad = EPAD - E
    src = jnp.concatenate([ei[0], jnp.zeros((pad,), jnp.int32)])
    dummy = N + (jnp.arange(pad, dtype=jnp.int32) % (NP - N))
    dst = jnp.concatenate([ei[1], dummy])
    dst2 = dst.reshape(NW * NCH, C)
    b1r = b1.reshape(1, D)
    b2r = b2.reshape(1, D)

    degp = _sc_deg(dst2)
    g1 = _tc1(x, W1, degp)
    acc1 = _sc_agg(g1, src, dst2)
    g2 = _tc2(acc1, g1, degp, W2, b1r)
    acc2 = _sc_agg(g2, src, dst2)
    return _tc3(acc2, g2, degp, b2r)


# speedup vs baseline: 1.0573x; 1.0573x over previous
import functools

import jax
import jax.numpy as jnp
from jax import lax
from jax.experimental import pallas as pl
from jax.experimental.pallas import tpu as pltpu
from jax.experimental.pallas import tpu_sc as plsc

N = 10000
D = 128
E = 320000

NC = 2
NS = 16
NW = NC * NS

C = 128
NB = 2

NCH0 = 124
NCH1 = 36
EP0 = NCH0 * C
EP1 = NCH1 * C
EPAD = NS * (EP0 + EP1)
NROW1 = NS * NCH0

NCHD = EPAD // (NW * C)
EPD = NCHD * C

NP = 10240
ZR = NP // NS

_mesh = plsc.VectorSubcoreMesh(core_axis_name="c", subcore_axis_name="s")
_sc_params = pltpu.CompilerParams(use_tc_tiling_on_sc=False)


def _fill(ref, nrows, ncols, value):
    v = jnp.full((16,), value, jnp.float32)

    def body(i, carry):
        for c in range(ncols // 16):
            ref[i, pl.ds(c * 16, 16)] = v
        return carry

    lax.fori_loop(0, nrows, body, 0)


@functools.partial(
    pl.kernel,
    out_type=jax.ShapeDtypeStruct((NC, NP, 16), jnp.float32),
    mesh=_mesh,
    scratch_types=[
        pltpu.VMEM((NCHD, C), jnp.int32),
        pltpu.VMEM((C, 16), jnp.float32),
        pltpu.VMEM_SHARED((NP, 16), jnp.float32),
    ],
    compiler_params=_sc_params,
)
def _sc_deg(dst_hbm, out_hbm, didx, ones_v, acc):
    cid = lax.axis_index("c")
    sid = lax.axis_index("s")
    wid = cid * NS + sid

    _fill(ones_v, C, 16, 0.0)
    zbase = sid * ZR
    for k in range(ZR // C):
        pltpu.sync_copy(ones_v, acc.at[pl.ds(zbase + k * C, C)])
    _fill(ones_v, C, 16, 1.0)
    pltpu.sync_copy(dst_hbm.at[pl.ds(wid * NCHD, NCHD)], didx)
    plsc.subcore_barrier()

    def chunk(i, carry):
        pltpu.sync_copy(ones_v, acc.at[didx.at[i]], add=True)
        return carry

    lax.fori_loop(0, NCHD, chunk, 0)
    plsc.subcore_barrier()

    pltpu.sync_copy(acc.at[pl.ds(zbase, ZR)],
                    out_hbm.at[cid, pl.ds(zbase, ZR)])


@functools.partial(
    pl.kernel,
    out_type=jax.ShapeDtypeStruct((NC, NP, D), jnp.float32),
    mesh=_mesh,
    scratch_types=[
        pltpu.VMEM((EP0,), jnp.int32),
        [pltpu.VMEM((1, C), jnp.int32) for _ in range(NB)],
        [pltpu.VMEM((C, D), jnp.float32) for _ in range(NB)],
        pltpu.VMEM_SHARED((NP, D), jnp.float32),
        [pltpu.SemaphoreType.DMA for _ in range(NB)],
        [pltpu.SemaphoreType.DMA for _ in range(NB)],
    ],
    compiler_params=_sc_params,
)
def _sc_agg(g_hbm, src_hbm, dst_hbm, out_hbm, sidx, didxs, rows,
            acc, sems, dsems):
    cid = lax.axis_index("c")
    sid = lax.axis_index("s")

    _fill(rows[0], C, D, 0.0)
    zbase = sid * ZR
    for k in range(ZR // C):
        pltpu.sync_copy(rows[0], acc.at[pl.ds(zbase + k * C, C)])

    @pl.when(cid == 0)
    def _():
        pltpu.sync_copy(src_hbm.at[pl.ds(sid * EP0, EP0)], sidx)

    @pl.when(cid == 1)
    def _():
        pltpu.sync_copy(src_hbm.at[pl.ds(NS * EP0 + sid * EP1, EP1)],
                        sidx.at[pl.ds(0, EP1)])

    plsc.subcore_barrier()

    nch = jnp.where(cid == 0, NCH0, NCH1)
    drow = jnp.where(cid == 0, sid * NCH0, NROW1 + sid * NCH1)

    def gather(c, q):
        pltpu.async_copy(g_hbm.at[sidx.at[pl.ds(c * C, C)]], rows[q], sems[q])

    def gather_wait(c, q):
        pltpu.make_async_copy(g_hbm.at[sidx.at[pl.ds(c * C, C)]], rows[q],
                              sems[q]).wait()

    def didx_load(c, q):
        pltpu.async_copy(dst_hbm.at[drow + c], didxs[q].at[0], dsems[q])

    def didx_wait(c, q):
        pltpu.make_async_copy(dst_hbm.at[drow + c], didxs[q].at[0],
                              dsems[q]).wait()

    for q in range(NB):
        didx_load(q, q)
        gather(q, q)

    def rounds(p, carry):
        c0 = NB * p
        for q in range(NB):
            c = c0 + q
            gather_wait(c, q)
            didx_wait(c, q)
            pltpu.sync_copy(rows[q], acc.at[didxs[q].at[0]], add=True)
            nxt = jnp.minimum(c + NB, nch - 1)
            didx_load(nxt, q)
            gather(nxt, q)
        return carry

    lax.fori_loop(0, nch // NB, rounds, 0)
    for q in range(NB):
        gather_wait(nch - 1, q)
        didx_wait(nch - 1, q)
    plsc.subcore_barrier()

    pltpu.sync_copy(acc.at[pl.ds(zbase, ZR)],
                    out_hbm.at[cid, pl.ds(zbase, ZR)])


_RB = 1000


def _dis_of(dref):
    deg = dref[0, :, 0:1] + dref[1, :, 0:1] + 1.0
    return lax.rsqrt(deg)


def _tc1_body(x_ref, w_ref, d_ref, o_ref):
    dis = _dis_of(d_ref)
    h = jnp.dot(x_ref[...], w_ref[...], preferred_element_type=jnp.float32)
    o_ref[...] = h * dis


def _tc2_body(a_ref, g_ref, d_ref, w_ref, b_ref, o_ref):
    dis = _dis_of(d_ref)
    s = a_ref[0] + a_ref[1] + g_ref[...]
    r = jnp.maximum(s * dis + b_ref[...], 0.0)
    o_ref[...] = jnp.dot(r, w_ref[...], preferred_element_type=jnp.float32) * dis


def _tc3_body(a_ref, g_ref, d_ref, b_ref, o_ref):
    dis = _dis_of(d_ref)
    s = a_ref[0] + a_ref[1] + g_ref[...]
    o_ref[...] = s * dis + b_ref[...]


def _row_spec(i):
    return (i, 0)


_spec_rows = pl.BlockSpec((_RB, D), _row_spec)
_spec_acc = pl.BlockSpec((NC, _RB, D), lambda i: (0, i, 0))
_spec_deg = pl.BlockSpec((NC, _RB, 16), lambda i: (0, i, 0))
_spec_w = pl.BlockSpec((D, D), lambda i: (0, 0))
_spec_b = pl.BlockSpec((1, D), lambda i: (0, 0))

_GRID = (N // _RB,)
_out_rows = jax.ShapeDtypeStruct((N, D), jnp.float32)

_tc1 = pl.pallas_call(
    _tc1_body, grid=_GRID,
    in_specs=[_spec_rows, _spec_w, _spec_deg],
    out_specs=_spec_rows, out_shape=_out_rows)

_tc2 = pl.pallas_call(
    _tc2_body, grid=_GRID,
    in_specs=[_spec_acc, _spec_rows, _spec_deg, _spec_w, _spec_b],
    out_specs=_spec_rows, out_shape=_out_rows)

_tc3 = pl.pallas_call(
    _tc3_body, grid=_GRID,
    in_specs=[_spec_acc, _spec_rows, _spec_deg, _spec_b],
    out_specs=_spec_rows, out_shape=_out_rows)


def kernel(x, edge_index, W1, b1, W2, b2):
    ei = edge_index.astype(jnp.int32)
    pad = EPAD - E
    e0 = NS * EP0 - pad
    dummy = N + (jnp.arange(pad, dtype=jnp.int32) % (NP - N))
    zpad = jnp.zeros((pad,), jnp.int32)
    src = jnp.concatenate([ei[0, :e0], zpad, ei[0, e0:]])
    dst = jnp.concatenate([ei[1, :e0], dummy, ei[1, e0:]])
    dst2 = dst.reshape(EPAD // C, C)
    b1r = b1.reshape(1, D)
    b2r = b2.reshape(1, D)

    degp = _sc_deg(dst2)
    g1 = _tc1(x, W1, degp)
    acc1 = _sc_agg(g1, src, dst2)
    g2 = _tc2(acc1, g1, degp, W2, b1r)
    acc2 = _sc_agg(g2, src, dst2)
    return _tc3(acc2, g2, degp, b2r)


# speedup vs baseline: 1.1146x; 1.0542x over previous
import functools

import jax
import jax.numpy as jnp
from jax import lax
from jax.experimental import pallas as pl
from jax.experimental.pallas import tpu as pltpu
from jax.experimental.pallas import tpu_sc as plsc

N = 10000
D = 128
E = 320000

NC = 2
NS = 16
NW = NC * NS

C = 128
NB = 2

NCH0 = 124
NCH1 = 36
EP0 = NCH0 * C
EP1 = NCH1 * C
EPAD = NS * (EP0 + EP1)
NROW1 = NS * NCH0

NCHD = EPAD // (NW * C)
EPD = NCHD * C

NP = 10240
ZR = NP // NS

_mesh = plsc.VectorSubcoreMesh(core_axis_name="c", subcore_axis_name="s")
_sc_params = pltpu.CompilerParams(use_tc_tiling_on_sc=False)


def _fill(ref, nrows, ncols, value):
    v = jnp.full((16,), value, jnp.float32)

    def body(i, carry):
        for c in range(ncols // 16):
            ref[i, pl.ds(c * 16, 16)] = v
        return carry

    lax.fori_loop(0, nrows, body, 0)


@functools.partial(
    pl.kernel,
    out_type=jax.ShapeDtypeStruct((NC, NP, 16), jnp.float32),
    mesh=_mesh,
    scratch_types=[
        pltpu.VMEM((NCHD, C), jnp.int32),
        pltpu.VMEM((C, 16), jnp.float32),
        pltpu.VMEM_SHARED((NP, 16), jnp.float32),
    ],
    compiler_params=_sc_params,
)
def _sc_deg(dst_hbm, out_hbm, didx, ones_v, acc):
    cid = lax.axis_index("c")
    sid = lax.axis_index("s")
    wid = cid * NS + sid

    _fill(ones_v, C, 16, 0.0)
    zbase = sid * ZR
    for k in range(ZR // C):
        pltpu.sync_copy(ones_v, acc.at[pl.ds(zbase + k * C, C)])
    _fill(ones_v, C, 16, 1.0)
    pltpu.sync_copy(dst_hbm.at[pl.ds(wid * NCHD, NCHD)], didx)
    plsc.subcore_barrier()

    def chunk(i, carry):
        pltpu.sync_copy(ones_v, acc.at[didx.at[i]], add=True)
        return carry

    lax.fori_loop(0, NCHD, chunk, 0)
    plsc.subcore_barrier()

    pltpu.sync_copy(acc.at[pl.ds(zbase, ZR)],
                    out_hbm.at[cid, pl.ds(zbase, ZR)])


@functools.partial(
    pl.kernel,
    out_type=jax.ShapeDtypeStruct((NC, NP, D), jnp.float32),
    mesh=_mesh,
    scratch_types=[
        pltpu.VMEM((EP0,), jnp.int32),
        [pltpu.VMEM((1, C), jnp.int32) for _ in range(NB)],
        [pltpu.VMEM((C, D), jnp.float32) for _ in range(NB)],
        pltpu.VMEM_SHARED((NP, D), jnp.float32),
        [pltpu.SemaphoreType.DMA for _ in range(NB)],
        [pltpu.SemaphoreType.DMA for _ in range(NB)],
    ],
    compiler_params=_sc_params,
)
def _sc_agg(g_hbm, src_hbm, dst_hbm, out_hbm, sidx, didxs, rows,
            acc, sems, dsems):
    cid = lax.axis_index("c")
    sid = lax.axis_index("s")

    _fill(rows[0], C, D, 0.0)
    zbase = sid * ZR
    for k in range(ZR // C):
        pltpu.sync_copy(rows[0], acc.at[pl.ds(zbase + k * C, C)])

    @pl.when(cid == 0)
    def _():
        pltpu.sync_copy(src_hbm.at[pl.ds(sid * EP0, EP0)], sidx)

    @pl.when(cid == 1)
    def _():
        pltpu.sync_copy(src_hbm.at[pl.ds(NS * EP0 + sid * EP1, EP1)],
                        sidx.at[pl.ds(0, EP1)])

    plsc.subcore_barrier()

    nch = jnp.where(cid == 0, NCH0, NCH1)
    drow = jnp.where(cid == 0, sid * NCH0, NROW1 + sid * NCH1)

    def gather(c, q):
        pltpu.async_copy(g_hbm.at[sidx.at[pl.ds(c * C, C)]], rows[q], sems[q])

    def gather_wait(c, q):
        pltpu.make_async_copy(g_hbm.at[sidx.at[pl.ds(c * C, C)]], rows[q],
                              sems[q]).wait()

    def didx_load(c, q):
        pltpu.async_copy(dst_hbm.at[drow + c], didxs[q].at[0], dsems[q])

    def didx_wait(c, q):
        pltpu.make_async_copy(dst_hbm.at[drow + c], didxs[q].at[0],
                              dsems[q]).wait()

    for q in range(NB):
        didx_load(q, q)
        gather(q, q)

    def rounds(p, carry):
        c0 = NB * p
        for q in range(NB):
            c = c0 + q
            gather_wait(c, q)
            didx_wait(c, q)
            pltpu.sync_copy(rows[q], acc.at[didxs[q].at[0]], add=True)
            didx_load(c + NB, q)
            gather(c + NB, q)
        return carry

    lax.fori_loop(0, nch // NB - 1, rounds, 0)
    for q in range(NB):
        c = nch - NB + q
        gather_wait(c, q)
        didx_wait(c, q)
        pltpu.sync_copy(rows[q], acc.at[didxs[q].at[0]], add=True)
    plsc.subcore_barrier()

    pltpu.sync_copy(acc.at[pl.ds(zbase, ZR)],
                    out_hbm.at[cid, pl.ds(zbase, ZR)])


_RB = 1000


def _dis_of(dref):
    deg = dref[0, :, 0:1] + dref[1, :, 0:1] + 1.0
    return lax.rsqrt(deg)


def _tc1_body(x_ref, w_ref, d_ref, o_ref):
    dis = _dis_of(d_ref)
    h = jnp.dot(x_ref[...], w_ref[...], preferred_element_type=jnp.float32)
    o_ref[...] = h * dis


def _tc2_body(a_ref, g_ref, d_ref, w_ref, b_ref, o_ref):
    dis = _dis_of(d_ref)
    s = a_ref[0] + a_ref[1] + g_ref[...]
    r = jnp.maximum(s * dis + b_ref[...], 0.0)
    o_ref[...] = jnp.dot(r, w_ref[...], preferred_element_type=jnp.float32) * dis


def _tc3_body(a_ref, g_ref, d_ref, b_ref, o_ref):
    dis = _dis_of(d_ref)
    s = a_ref[0] + a_ref[1] + g_ref[...]
    o_ref[...] = s * dis + b_ref[...]


def _row_spec(i):
    return (i, 0)


_spec_rows = pl.BlockSpec((_RB, D), _row_spec)
_spec_acc = pl.BlockSpec((NC, _RB, D), lambda i: (0, i, 0))
_spec_deg = pl.BlockSpec((NC, _RB, 16), lambda i: (0, i, 0))
_spec_w = pl.BlockSpec((D, D), lambda i: (0, 0))
_spec_b = pl.BlockSpec((1, D), lambda i: (0, 0))

_GRID = (N // _RB,)
_out_rows = jax.ShapeDtypeStruct((N, D), jnp.float32)

_tc1 = pl.pallas_call(
    _tc1_body, grid=_GRID,
    in_specs=[_spec_rows, _spec_w, _spec_deg],
    out_specs=_spec_rows, out_shape=_out_rows)

_tc2 = pl.pallas_call(
    _tc2_body, grid=_GRID,
    in_specs=[_spec_acc, _spec_rows, _spec_deg, _spec_w, _spec_b],
    out_specs=_spec_rows, out_shape=_out_rows)

_tc3 = pl.pallas_call(
    _tc3_body, grid=_GRID,
    in_specs=[_spec_acc, _spec_rows, _spec_deg, _spec_b],
    out_specs=_spec_rows, out_shape=_out_rows)


def kernel(x, edge_index, W1, b1, W2, b2):
    ei = edge_index.astype(jnp.int32)
    pad = EPAD - E
    e0 = NS * EP0 - pad
    dummy = N + (jnp.arange(pad, dtype=jnp.int32) % (NP - N))
    zpad = jnp.zeros((pad,), jnp.int32)
    src = jnp.concatenate([ei[0, :e0], zpad, ei[0, e0:]])
    dst = jnp.concatenate([ei[1, :e0], dummy, ei[1, e0:]])
    dst2 = dst.reshape(EPAD // C, C)
    b1r = b1.reshape(1, D)
    b2r = b2.reshape(1, D)

    degp = _sc_deg(dst2)
    g1 = _tc1(x, W1, degp)
    acc1 = _sc_agg(g1, src, dst2)
    g2 = _tc2(acc1, g1, degp, W2, b1r)
    acc2 = _sc_agg(g2, src, dst2)
    return _tc3(acc2, g2, degp, b2r)
